# Initial kernel scaffold; baseline (speedup 1.0000x reference)
import jax
import jax.numpy as jnp
from jax.experimental import pallas as pl


def kernel(x, neighbors, times, t, Wk, bk, Wq, bq, Wv, bv, w0, b0, Wt, Bt, Tk, btk, Tq, btq, Tv, btv, W1, b1, W2, b2):
    raise NotImplementedError("write your pallas kernel here")


# speedup vs baseline: 2.7016x; 2.7016x over previous
import functools

import jax
import jax.numpy as jnp
from jax import lax
from jax.experimental import pallas as pl
from jax.experimental.pallas import tpu as pltpu
from jax.experimental.pallas import tpu_sc as plsc

N = 10000
DEG = 32
F = 128
TDIM = 16

NW = 32
NODES_PER_W = 320
NPAD = NW * NODES_PER_W
CH = 4
ROWS = CH * DEG
CHUNKS = NODES_PER_W // CH


def _sc_gather_sum(x, nbr_flat):
    info = plsc.get_sparse_core_info()
    nc = info.num_cores

    mesh = plsc.VectorSubcoreMesh(core_axis_name="c", subcore_axis_name="s")

    @functools.partial(
        pl.kernel,
        mesh=mesh,
        out_type=jax.ShapeDtypeStruct((NPAD, F), jnp.float32),
        scratch_types=[
            pltpu.VMEM((ROWS,), jnp.int32),
            pltpu.VMEM((ROWS,), jnp.int32),
            pltpu.VMEM((ROWS, F), jnp.float32),
            pltpu.VMEM((CH, F), jnp.float32),
            pltpu.VMEM_SHARED((NW * CH, F), jnp.float32),
            pltpu.SemaphoreType.DMA,
        ],
    )
    def body(x_hbm, nbr_hbm, out_hbm, idx_v, dst_v, rows_v, zeros_v, acc_sh, sem):
        wid = lax.axis_index("s") * nc + lax.axis_index("c")
        abase = wid * CH
        for j in range(ROWS // 16):
            dst_v[pl.ds(j * 16, 16)] = jnp.full((16,), j * 16 // DEG, jnp.int32) + abase
        z16 = jnp.zeros((16,), jnp.float32)
        for r in range(CH):
            for cidx in range(F // 16):
                zeros_v[r, pl.ds(cidx * 16, 16)] = z16

        def chunk(i, carry):
            base = wid * NODES_PER_W + i * CH
            pltpu.sync_copy(nbr_hbm.at[pl.ds(base * DEG, ROWS)], idx_v)
            cp = pltpu.async_copy(x_hbm.at[idx_v], rows_v, sem)
            pltpu.sync_copy(zeros_v, acc_sh.at[pl.ds(abase, CH)])
            cp.wait()
            pltpu.sync_copy(rows_v, acc_sh.at[dst_v], add=True)
            pltpu.sync_copy(acc_sh.at[pl.ds(abase, CH)], out_hbm.at[pl.ds(base, CH)])
            return carry

        lax.fori_loop(0, CHUNKS, chunk, 0)

    return body(x, nbr_flat)


BLK = 400


def _tc_body(scal_ref, times_ref, x_ref, g_ref, Wv_ref, Tv_ref, W1a_ref,
             W1b_ref, W2_ref, b1_ref, b2_ref, o_ref):
    times = times_ref[...]
    w0 = scal_ref[0, 0]
    b0 = scal_ref[0, 1]
    rs = jnp.sum(times, axis=1, keepdims=True)
    acc = (rs * w0 + (DEG * b0)) * Tv_ref[0:1, :]
    for j in range(TDIM - 1):
        wt = scal_ref[0, 2 + j]
        bt = scal_ref[0, 2 + (TDIM - 1) + j]
        zj = jnp.sum(jnp.sin(times * wt + bt), axis=1, keepdims=True)
        acc = acc + zj * Tv_ref[j + 1:j + 2, :]
    h = jnp.dot(g_ref[...], Wv_ref[...], preferred_element_type=jnp.float32) + acc
    pre = (jnp.dot(x_ref[...], W1a_ref[...], preferred_element_type=jnp.float32)
           + jnp.dot(h, W1b_ref[...], preferred_element_type=jnp.float32)
           + b1_ref[...])
    o_ref[...] = (jnp.dot(jnp.maximum(pre, 0.0), W2_ref[...],
                          preferred_element_type=jnp.float32) + b2_ref[...])


def _tc_post(scal, times, x, g, Wv, Tv, W1a, W1b, W2, b1eff, b2r):
    grid = (N // BLK,)
    full = lambda shape: pl.BlockSpec(shape, lambda i: (0, 0))
    return pl.pallas_call(
        _tc_body,
        grid=grid,
        in_specs=[
            pl.BlockSpec(memory_space=pltpu.SMEM),
            pl.BlockSpec((BLK, DEG), lambda i: (i, 0)),
            pl.BlockSpec((BLK, F), lambda i: (i, 0)),
            pl.BlockSpec((BLK, F), lambda i: (i, 0)),
            full((F, F)),
            full((TDIM, F)),
            full((F, F)),
            full((F, F)),
            full((F, F)),
            full((1, F)),
            full((1, F)),
        ],
        out_specs=pl.BlockSpec((BLK, F), lambda i: (i, 0)),
        out_shape=jax.ShapeDtypeStruct((N, F), jnp.float32),
    )(scal, times, x, g, Wv, Tv, W1a, W1b, W2, b1eff, b2r)


def kernel(x, neighbors, times, t, Wk, bk, Wq, bq, Wv, bv, w0, b0, Wt, Bt,
           Tk, btk, Tq, btq, Tv, btv, W1, b1, W2, b2):
    nbr = neighbors.astype(jnp.int32).reshape(-1)
    nbr_flat = jnp.pad(nbr, (0, (NPAD - N) * DEG))
    g = _sc_gather_sum(x, nbr_flat)[:N]

    scal = jnp.concatenate(
        [w0.reshape(1), b0.reshape(1), Wt.reshape(TDIM - 1), Bt.reshape(TDIM - 1)]
    ).reshape(1, 2 * TDIM)
    W1a = W1[:F]
    W1b = W1[F:]
    b1eff = (b1 + (DEG * (bv + btv)) @ W1b).reshape(1, F)
    return _tc_post(scal, times, x, g, Wv, Tv, W1a, W1b, W2, b1eff,
                    b2.reshape(1, F))


# speedup vs baseline: 3.9031x; 1.4447x over previous
import functools

import jax
import jax.numpy as jnp
from jax import lax
from jax.experimental import pallas as pl
from jax.experimental.pallas import tpu as pltpu
from jax.experimental.pallas import tpu_sc as plsc

N = 10000
DEG = 32
F = 128
TDIM = 16

NW = 32
NODES_PER_W = 320
NPAD = NW * NODES_PER_W
CH = 4
ROWS = CH * DEG
CHUNKS = NODES_PER_W // CH
IDX_PER_W = NODES_PER_W * DEG


def _sc_gather_sum(x, nbr_flat):
    info = plsc.get_sparse_core_info()
    nc = info.num_cores

    mesh = plsc.VectorSubcoreMesh(core_axis_name="c", subcore_axis_name="s")

    @functools.partial(
        pl.kernel,
        mesh=mesh,
        out_type=jax.ShapeDtypeStruct((NPAD, F), jnp.float32),
        scratch_types=[
            pltpu.VMEM((CHUNKS, ROWS), jnp.int32),
            pltpu.VMEM((ROWS, F), jnp.float32),
            pltpu.VMEM((ROWS, F), jnp.float32),
            pltpu.VMEM((CH, F), jnp.float32),
            pltpu.VMEM((CH, F), jnp.float32),
            pltpu.SemaphoreType.DMA,
            pltpu.SemaphoreType.DMA,
            pltpu.SemaphoreType.DMA,
            pltpu.SemaphoreType.DMA,
        ],
    )
    def body(x_hbm, nbr_hbm, out_hbm, idx_v, rows0, rows1, pool0, pool1,
             sem0, sem1, osem0, osem1):
        wid = lax.axis_index("s") * nc + lax.axis_index("c")

        pltpu.sync_copy(nbr_hbm.at[pl.ds(wid * CHUNKS, CHUNKS)], idx_v)

        rows = (rows0, rows1)
        pools = (pool0, pool1)
        sems = (sem0, sem1)
        osems = (osem0, osem1)

        def desc(chunk_i, slot):
            ci = lax.min(chunk_i, CHUNKS - 1)
            return pltpu.make_async_copy(
                x_hbm.at[idx_v.at[ci]], rows[slot], sems[slot])

        def odesc(chunk_i, slot):
            base = wid * NODES_PER_W + lax.min(chunk_i, CHUNKS - 1) * CH
            return pltpu.make_async_copy(
                pools[slot], out_hbm.at[pl.ds(base, CH)], osems[slot])

        desc(0, 0).start()
        desc(1, 1).start()

        def pair(i2, carry):
            for b in range(2):
                i = i2 * 2 + b
                desc(i, b).wait()
                rb = rows[b]
                pb = pools[b]
                @pl.when(i2 > 0)
                def _():
                    odesc(i - 2, b).wait()
                for c in range(CH):
                    for col in range(F // 16):
                        sl = pl.ds(col * 16, 16)
                        vals = [rb[c * DEG + d, sl] for d in range(DEG)]
                        while len(vals) > 1:
                            vals = [vals[k] + vals[k + 1]
                                    for k in range(0, len(vals) - 1, 2)] + (
                                        [vals[-1]] if len(vals) % 2 else [])
                        pb[c, sl] = vals[0]
                odesc(i, b).start()
                desc(i + 2, b).start()
            return carry

        lax.fori_loop(0, CHUNKS // 2, pair, 0)
        desc(CHUNKS, 0).wait()
        desc(CHUNKS + 1, 1).wait()
        odesc(CHUNKS - 2, 0).wait()
        odesc(CHUNKS - 1, 1).wait()

    return body(x, nbr_flat)


BLK = 512
_PREC = lax.Precision.HIGHEST


def _tc_body(scal_ref, tt_ref, x_ref, g_ref, Wv_ref, Tv_ref, W1a_ref,
             W1b_ref, W2_ref, b1_ref, b2_ref, o_ref):
    tt = tt_ref[...]
    w0 = scal_ref[0, 0]
    b0 = scal_ref[0, 1]
    zrows = [jnp.sum(tt, axis=0, keepdims=True) * w0 + (DEG * b0)]
    for j in range(TDIM - 1):
        wt = scal_ref[0, 2 + j]
        bt = scal_ref[0, 2 + (TDIM - 1) + j]
        zrows.append(jnp.sum(jnp.sin(tt * wt + bt), axis=0, keepdims=True))
    Z = jnp.concatenate(zrows, axis=0)
    ht = lax.dot_general(Z, Tv_ref[...], (((0,), (0,)), ((), ())),
                         precision=_PREC, preferred_element_type=jnp.float32)
    h = jnp.dot(g_ref[...], Wv_ref[...], precision=_PREC,
                preferred_element_type=jnp.float32) + ht
    pre = (jnp.dot(x_ref[...], W1a_ref[...], precision=_PREC,
                   preferred_element_type=jnp.float32)
           + jnp.dot(h, W1b_ref[...], precision=_PREC,
                     preferred_element_type=jnp.float32)
           + b1_ref[...])
    o_ref[...] = (jnp.dot(jnp.maximum(pre, 0.0), W2_ref[...], precision=_PREC,
                          preferred_element_type=jnp.float32) + b2_ref[...])


def _tc_post(scal, times_t, x, g, Wv, Tv, W1a, W1b, W2, b1eff, b2r):
    grid = (pl.cdiv(N, BLK),)
    full = lambda shape: pl.BlockSpec(shape, lambda i: (0, 0))
    return pl.pallas_call(
        _tc_body,
        grid=grid,
        in_specs=[
            pl.BlockSpec(memory_space=pltpu.SMEM),
            pl.BlockSpec((DEG, BLK), lambda i: (0, i)),
            pl.BlockSpec((BLK, F), lambda i: (i, 0)),
            pl.BlockSpec((BLK, F), lambda i: (i, 0)),
            full((F, F)),
            full((TDIM, F)),
            full((F, F)),
            full((F, F)),
            full((F, F)),
            full((1, F)),
            full((1, F)),
        ],
        out_specs=pl.BlockSpec((BLK, F), lambda i: (i, 0)),
        out_shape=jax.ShapeDtypeStruct((N, F), jnp.float32),
    )(scal, times_t, x, g, Wv, Tv, W1a, W1b, W2, b1eff, b2r)


def kernel(x, neighbors, times, t, Wk, bk, Wq, bq, Wv, bv, w0, b0, Wt, Bt,
           Tk, btk, Tq, btq, Tv, btv, W1, b1, W2, b2):
    nbr = neighbors.astype(jnp.int32).reshape(-1)
    nbr_flat = jnp.pad(nbr, (0, (NPAD - N) * DEG)).reshape(NPAD // CH, ROWS)
    g = _sc_gather_sum(x, nbr_flat)[:N]

    times_t = jnp.pad(times.T, ((0, 0), (0, BLK * pl.cdiv(N, BLK) - N)))
    scal = jnp.concatenate(
        [w0.reshape(1), b0.reshape(1), Wt.reshape(TDIM - 1), Bt.reshape(TDIM - 1)]
    ).reshape(1, 2 * TDIM)
    W1a = W1[:F]
    W1b = W1[F:]
    b1eff = (b1 + (DEG * (bv + btv)) @ W1b).reshape(1, F)
    return _tc_post(scal, times_t, x, g, Wv, Tv, W1a, W1b, W2, b1eff,
                    b2.reshape(1, F))
